# Initial kernel scaffold; baseline (speedup 1.0000x reference)
import jax
import jax.numpy as jnp
from jax.experimental import pallas as pl


def kernel(x_binned, tables):
    raise NotImplementedError("write your pallas kernel here")


# speedup vs baseline: 1.1477x; 1.1477x over previous
import jax
import jax.numpy as jnp
from jax import lax
from jax.experimental import pallas as pl
from jax.experimental.pallas import tpu as pltpu
from jax.experimental.pallas import tpu_sc as plsc

_NUM_FIELDS = 26
_VOCAB = 100000
_DIM = 32
_BATCH = 16384

_NC = 2
_NS = 16
_NW = _NC * _NS
_TOTAL = _BATCH * _NUM_FIELDS
_PER_W = _TOTAL // _NW
_CHUNK = 128
_NCHUNK = _PER_W // _CHUNK
_PERIOD = 13
_REPS = (_PER_W // 16) // _PERIOD


def _sc_body(x_hbm, tab_hbm, out_hbm, idx_v, offs_v, rows_v, gsem):
    wid = lax.axis_index("s") * _NC + lax.axis_index("c")
    base = pl.multiple_of(wid * _PER_W, _PER_W)

    pltpu.sync_copy(x_hbm.at[pl.ds(base, _PER_W)], idx_v)

    def build(t, c):
        lane = lax.iota(jnp.int32, 16) + t * 16
        o = pl.multiple_of(t * 16, 16)
        offs_v[pl.ds(o, 16)] = lax.rem(lane, _NUM_FIELDS) * _VOCAB
        return c

    lax.fori_loop(0, _PERIOD, build, 0)

    def rep(r, c):
        def inner(t, c2):
            s = pl.multiple_of((r * _PERIOD + t) * 16, 16)
            o = pl.multiple_of(t * 16, 16)
            idx_v[pl.ds(s, 16)] = idx_v[pl.ds(s, 16)] + offs_v[pl.ds(o, 16)]
            return c2

        return lax.fori_loop(0, _PERIOD, inner, c)

    lax.fori_loop(0, _REPS, rep, 0)

    def chunk(j, c):
        s = pl.multiple_of(j * _CHUNK, _CHUNK)
        pltpu.async_copy(tab_hbm.at[idx_v.at[pl.ds(s, _CHUNK)]], rows_v, gsem).wait()
        pltpu.sync_copy(rows_v, out_hbm.at[pl.ds(base + s, _CHUNK)])
        return c

    lax.fori_loop(0, _NCHUNK, chunk, 0)


@jax.jit
def _binned_embed(x_flat, tab_flat):
    mesh = plsc.VectorSubcoreMesh(core_axis_name="c", subcore_axis_name="s")
    f = pl.kernel(
        _sc_body,
        out_type=jax.ShapeDtypeStruct((_TOTAL, _DIM), jnp.float32),
        mesh=mesh,
        scratch_types=[
            pltpu.VMEM((_PER_W,), jnp.int32),
            pltpu.VMEM((_PERIOD * 16,), jnp.int32),
            pltpu.VMEM((_CHUNK, _DIM), jnp.float32),
            pltpu.SemaphoreType.DMA,
        ],
        compiler_params=pltpu.CompilerParams(use_tc_tiling_on_sc=False),
    )
    return f(x_flat, tab_flat)


def kernel(x_binned, tables):
    x_flat = x_binned.reshape(-1)
    tab_flat = tables.reshape(_NUM_FIELDS * _VOCAB, _DIM)
    out = _binned_embed(x_flat, tab_flat)
    return out.reshape(_BATCH, _NUM_FIELDS * _DIM)


# speedup vs baseline: 1.2143x; 1.0580x over previous
import jax
import jax.numpy as jnp
from jax import lax
from jax.experimental import pallas as pl
from jax.experimental.pallas import tpu as pltpu
from jax.experimental.pallas import tpu_sc as plsc

_NUM_FIELDS = 26
_VOCAB = 100000
_DIM = 32
_BATCH = 16384

_NC = 2
_NS = 16
_NW = _NC * _NS
_TOTAL = _BATCH * _NUM_FIELDS
_PER_W = _TOTAL // _NW
_CHUNK = 128
_NCHUNK = _PER_W // _CHUNK
_PERIOD = 13
_K = 13
_NBLK = _NCHUNK // _K
_VPB = _K * _CHUNK // 16
_RPB = _VPB // _PERIOD


def _sc_body(x_hbm, tab_hbm, out_hbm, idx_v, offs_v, rows_v, gsem, wsem):
    wid = lax.axis_index("s") * _NC + lax.axis_index("c")
    base = pl.multiple_of(wid * _PER_W, _PER_W)

    pltpu.sync_copy(x_hbm.at[pl.ds(base, _PER_W)], idx_v)

    def build(t, c):
        lane = lax.iota(jnp.int32, 16) + t * 16
        o = pl.multiple_of(t * 16, 16)
        offs_v[pl.ds(o, 16)] = lax.rem(lane, _NUM_FIELDS) * _VOCAB
        return c

    lax.fori_loop(0, _PERIOD, build, 0)

    def _add_block_offsets(blk):
        vb = blk * _VPB

        def rep(r, c):
            def inner(t, c2):
                s = pl.multiple_of((vb + r * _PERIOD + t) * 16, 16)
                o = pl.multiple_of(t * 16, 16)
                idx_v[pl.ds(s, 16)] = idx_v[pl.ds(s, 16)] + offs_v[pl.ds(o, 16)]
                return c2

            return lax.fori_loop(0, _PERIOD, inner, c)

        lax.fori_loop(0, _RPB, rep, 0)

    _add_block_offsets(0)

    def block(j, c):
        cbase = pl.multiple_of(j * _K * _CHUNK, _CHUNK)
        gds = []
        for b in range(_K):
            s = pl.multiple_of(cbase + b * _CHUNK, _CHUNK)
            gds.append(
                pltpu.async_copy(
                    tab_hbm.at[idx_v.at[pl.ds(s, _CHUNK)]], rows_v.at[b], gsem
                )
            )

        @pl.when(j + 1 < _NBLK)
        def _():
            _add_block_offsets(j + 1)

        wds = []
        for b in range(_K):
            gds[b].wait()
            s = pl.multiple_of(cbase + b * _CHUNK, _CHUNK)
            wds.append(
                pltpu.async_copy(
                    rows_v.at[b], out_hbm.at[pl.ds(base + s, _CHUNK)], wsem
                )
            )
        for b in range(_K):
            wds[b].wait()
        return c

    lax.fori_loop(0, _NBLK, block, 0)


@jax.jit
def _binned_embed(x_flat, tab_flat):
    mesh = plsc.VectorSubcoreMesh(core_axis_name="c", subcore_axis_name="s")
    f = pl.kernel(
        _sc_body,
        out_type=jax.ShapeDtypeStruct((_TOTAL, _DIM), jnp.float32),
        mesh=mesh,
        scratch_types=[
            pltpu.VMEM((_PER_W,), jnp.int32),
            pltpu.VMEM((_PERIOD * 16,), jnp.int32),
            pltpu.VMEM((_K, _CHUNK, _DIM), jnp.float32),
            pltpu.SemaphoreType.DMA,
            pltpu.SemaphoreType.DMA,
        ],
        compiler_params=pltpu.CompilerParams(use_tc_tiling_on_sc=False),
    )
    return f(x_flat, tab_flat)


def kernel(x_binned, tables):
    x_flat = x_binned.reshape(-1)
    tab_flat = tables.reshape(_NUM_FIELDS * _VOCAB, _DIM)
    out = _binned_embed(x_flat, tab_flat)
    return out.reshape(_BATCH, _NUM_FIELDS * _DIM)
